# speedup vs baseline: 1.1210x; 1.1210x over previous
import functools

import jax
import jax.numpy as jnp
from jax import lax
from jax.experimental import pallas as pl
from jax.experimental.pallas import tpu as pltpu
from jax.experimental.pallas import tpu_sc as plsc

_BATCH = 16384
_DIM = 128
_LANES = 16
_NC = 2
_NS = 16
_NW = _NC * _NS
_BPW = _BATCH // _NW
_CHUNK = 256
_NCHUNK = _BPW // _CHUNK


def _body(feat_hbm, lab_hbm, cent_hbm, out_hbm,
          idx_v, feat_v, rows_v, part_v, sem):
    wid = lax.axis_index("s") * _NC + lax.axis_index("c")
    base = wid * _BPW
    pltpu.sync_copy(lab_hbm.at[pl.ds(base, _BPW)], idx_v)

    zero = jnp.zeros((_LANES,), jnp.float32)
    accs = (zero,) * 8

    for chunk in range(_NCHUNK):
        rbase = chunk * _CHUNK
        cp = pltpu.async_copy(
            cent_hbm.at[idx_v.at[pl.ds(rbase, _CHUNK)]], rows_v, sem)
        pltpu.sync_copy(
            feat_hbm.at[pl.ds((base + rbase) * _DIM, _CHUNK * _DIM)], feat_v)
        cp.wait()

        def row(r, accs):
            out = []
            for c in range(8):
                f = feat_v[pl.ds(r * _DIM + c * _LANES, _LANES)]
                cv = rows_v[r, pl.ds(c * _LANES, _LANES)]
                d = f - cv
                out.append(accs[c] + d * d)
            return tuple(out)

        accs = lax.fori_loop(0, _CHUNK, row, accs)

    total = accs[0]
    for c in range(1, 8):
        total = total + accs[c]
    part_v[...] = total
    pltpu.sync_copy(part_v, out_hbm.at[wid])


@jax.jit
def kernel(features, labels, centers):
    feat_flat = features.reshape(_BATCH * _DIM)
    lab32 = labels.astype(jnp.int32)
    partials = pl.kernel(
        _body,
        out_type=jax.ShapeDtypeStruct((_NW, _LANES), jnp.float32),
        mesh=plsc.VectorSubcoreMesh(core_axis_name="c", subcore_axis_name="s"),
        scratch_types=[
            pltpu.VMEM((_BPW,), jnp.int32),
            pltpu.VMEM((_CHUNK * _DIM,), jnp.float32),
            pltpu.VMEM((_CHUNK, _DIM), jnp.float32),
            pltpu.VMEM((_LANES,), jnp.float32),
            pltpu.SemaphoreType.DMA,
        ],
    )(feat_flat, lab32, centers)
    return jnp.sum(partials) / (_BATCH * _DIM)


# speedup vs baseline: 1.1242x; 1.0028x over previous
import functools

import jax
import jax.numpy as jnp
from jax import lax
from jax.experimental import pallas as pl
from jax.experimental.pallas import tpu as pltpu
from jax.experimental.pallas import tpu_sc as plsc

_BATCH = 16384
_DIM = 128
_LANES = 16
_NC = 2
_NS = 16
_NW = _NC * _NS
_BPW = _BATCH // _NW
_CHUNK = 128
_NCHUNK = _BPW // _CHUNK
_UNROLL = 4
_NACC = 8


def _reduce_chunk(feat_v, rows_v, fbase, accs):
    def iter_body(i, accs):
        accs = list(accs)
        r = i * _UNROLL
        for u in range(_UNROLL):
            for c in range(_DIM // _LANES):
                f = feat_v[pl.ds(fbase + (r + u) * _DIM + c * _LANES, _LANES)]
                cv = rows_v[r + u, pl.ds(c * _LANES, _LANES)]
                d = f - cv
                a = (u * (_DIM // _LANES) + c) % _NACC
                accs[a] = accs[a] + d * d
        return tuple(accs)

    return lax.fori_loop(0, _CHUNK // _UNROLL, iter_body, accs)


def _body(feat_hbm, lab_hbm, cent_hbm, out_hbm,
          idx_v, feat_v, rows0, rows1, part_v, semf, sem0, sem1):
    wid = lax.axis_index("s") * _NC + lax.axis_index("c")
    base = wid * _BPW
    pltpu.sync_copy(lab_hbm.at[pl.ds(base, _BPW)], idx_v)
    fcp = pltpu.async_copy(
        feat_hbm.at[pl.ds(base * _DIM, _BPW * _DIM)], feat_v, semf)

    rows = (rows0, rows1)
    sems = (sem0, sem1)

    def gather(chunk):
        return pltpu.async_copy(
            cent_hbm.at[idx_v.at[pl.ds(chunk * _CHUNK, _CHUNK)]],
            rows[chunk % 2], sems[chunk % 2])

    cps = [None] * _NCHUNK
    cps[0] = gather(0)
    cps[1] = gather(1)
    fcp.wait()

    accs = (jnp.zeros((_LANES,), jnp.float32),) * _NACC
    for chunk in range(_NCHUNK):
        cps[chunk].wait()
        accs = _reduce_chunk(feat_v, rows[chunk % 2], chunk * _CHUNK * _DIM,
                             accs)
        if chunk + 2 < _NCHUNK:
            cps[chunk + 2] = gather(chunk + 2)

    total = accs[0]
    for c in range(1, _NACC):
        total = total + accs[c]
    part_v[...] = total
    pltpu.sync_copy(part_v, out_hbm.at[wid])


@jax.jit
def kernel(features, labels, centers):
    feat_flat = features.reshape(_BATCH * _DIM)
    lab32 = labels.astype(jnp.int32)
    partials = pl.kernel(
        _body,
        out_type=jax.ShapeDtypeStruct((_NW, _LANES), jnp.float32),
        mesh=plsc.VectorSubcoreMesh(core_axis_name="c", subcore_axis_name="s"),
        scratch_types=[
            pltpu.VMEM((_BPW,), jnp.int32),
            pltpu.VMEM((_BPW * _DIM,), jnp.float32),
            pltpu.VMEM((_CHUNK, _DIM), jnp.float32),
            pltpu.VMEM((_CHUNK, _DIM), jnp.float32),
            pltpu.VMEM((_LANES,), jnp.float32),
            pltpu.SemaphoreType.DMA,
            pltpu.SemaphoreType.DMA,
            pltpu.SemaphoreType.DMA,
        ],
    )(feat_flat, lab32, centers)
    return jnp.sum(partials) / (_BATCH * _DIM)


# speedup vs baseline: 1.1285x; 1.0038x over previous
import functools

import jax
import jax.numpy as jnp
from jax import lax
from jax.experimental import pallas as pl
from jax.experimental.pallas import tpu as pltpu
from jax.experimental.pallas import tpu_sc as plsc

_BATCH = 16384
_DIM = 128
_LANES = 16
_NC = 2
_NS = 16
_NW = _NC * _NS
_BPW = _BATCH // _NW
_CHUNK = 128
_NCHUNK = _BPW // _CHUNK
_UNROLL = 4
_NACC = 8


def _reduce_chunk(feat_v, rows_v, fbase, accs):
    def iter_body(i, accs):
        accs = list(accs)
        r = i * _UNROLL
        for u in range(_UNROLL):
            for c in range(_DIM // _LANES):
                f = feat_v[fbase + r + u, pl.ds(c * _LANES, _LANES)]
                cv = rows_v[r + u, pl.ds(c * _LANES, _LANES)]
                d = f - cv
                a = (u * (_DIM // _LANES) + c) % _NACC
                accs[a] = accs[a] + d * d
        return tuple(accs)

    return lax.fori_loop(0, _CHUNK // _UNROLL, iter_body, accs)


def _body(feat_hbm, lab_hbm, cent_hbm, out_hbm,
          idx_v, feat_v, rows0, rows1, part_v, semf, sem0, sem1):
    wid = lax.axis_index("s") * _NC + lax.axis_index("c")
    base = wid * _BPW
    pltpu.sync_copy(lab_hbm.at[pl.ds(base, _BPW)], idx_v)
    fcp = pltpu.async_copy(
        feat_hbm.at[pl.ds(base, _BPW), :], feat_v, semf)

    rows = (rows0, rows1)
    sems = (sem0, sem1)

    def gather(chunk):
        return pltpu.async_copy(
            cent_hbm.at[idx_v.at[pl.ds(chunk * _CHUNK, _CHUNK)]],
            rows[chunk % 2], sems[chunk % 2])

    cps = [None] * _NCHUNK
    cps[0] = gather(0)
    cps[1] = gather(1)
    fcp.wait()

    accs = (jnp.zeros((_LANES,), jnp.float32),) * _NACC
    for chunk in range(_NCHUNK):
        cps[chunk].wait()
        accs = _reduce_chunk(feat_v, rows[chunk % 2], chunk * _CHUNK, accs)
        if chunk + 2 < _NCHUNK:
            cps[chunk + 2] = gather(chunk + 2)

    total = accs[0]
    for c in range(1, _NACC):
        total = total + accs[c]
    part_v[...] = total
    pltpu.sync_copy(part_v, out_hbm.at[wid])


@jax.jit
def kernel(features, labels, centers):
    lab32 = labels.astype(jnp.int32)
    partials = pl.kernel(
        _body,
        out_type=jax.ShapeDtypeStruct((_NW, _LANES), jnp.float32),
        mesh=plsc.VectorSubcoreMesh(core_axis_name="c", subcore_axis_name="s"),
        scratch_types=[
            pltpu.VMEM((_BPW,), jnp.int32),
            pltpu.VMEM((_BPW, _DIM), jnp.float32),
            pltpu.VMEM((_CHUNK, _DIM), jnp.float32),
            pltpu.VMEM((_CHUNK, _DIM), jnp.float32),
            pltpu.VMEM((_LANES,), jnp.float32),
            pltpu.SemaphoreType.DMA,
            pltpu.SemaphoreType.DMA,
            pltpu.SemaphoreType.DMA,
        ],
    )(features, lab32, centers)
    return jnp.sum(partials) / (_BATCH * _DIM)


# speedup vs baseline: 1.1523x; 1.0211x over previous
import functools

import jax
import jax.numpy as jnp
from jax import lax
from jax.experimental import pallas as pl
from jax.experimental.pallas import tpu as pltpu
from jax.experimental.pallas import tpu_sc as plsc

_BATCH = 16384
_DIM = 128
_LANES = 16
_NC = 2
_NS = 16
_NW = _NC * _NS
_BPW = _BATCH // _NW
_CHUNK = 128
_NCHUNK = _BPW // _CHUNK
_UNROLL = 4
_NACC = 8


def _reduce_chunk(feat_v, rows_v, accs):
    def iter_body(i, accs):
        accs = list(accs)
        r = i * _UNROLL
        for u in range(_UNROLL):
            for c in range(_DIM // _LANES):
                f = feat_v[r + u, pl.ds(c * _LANES, _LANES)]
                cv = rows_v[r + u, pl.ds(c * _LANES, _LANES)]
                d = f - cv
                a = (u * (_DIM // _LANES) + c) % _NACC
                accs[a] = accs[a] + d * d
        return tuple(accs)

    return lax.fori_loop(0, _CHUNK // _UNROLL, iter_body, accs)


def _body(feat_hbm, lab_hbm, cent_hbm, out_hbm,
          idx_v, feat0, feat1, rows0, rows1, part_v, sem0, sem1):
    wid = lax.axis_index("s") * _NC + lax.axis_index("c")
    base = wid * _BPW
    pltpu.sync_copy(lab_hbm.at[pl.ds(base, _BPW)], idx_v)

    feats = (feat0, feat1)
    rows = (rows0, rows1)
    sems = (sem0, sem1)

    def fetch(chunk):
        s = chunk % 2
        cp_c = pltpu.async_copy(
            cent_hbm.at[idx_v.at[pl.ds(chunk * _CHUNK, _CHUNK)]],
            rows[s], sems[s])
        cp_f = pltpu.async_copy(
            feat_hbm.at[pl.ds(base + chunk * _CHUNK, _CHUNK), :],
            feats[s], sems[s])
        return (cp_c, cp_f)

    cps = [None] * _NCHUNK
    cps[0] = fetch(0)
    cps[1] = fetch(1)

    accs = (jnp.zeros((_LANES,), jnp.float32),) * _NACC
    for chunk in range(_NCHUNK):
        cps[chunk][0].wait()
        cps[chunk][1].wait()
        accs = _reduce_chunk(feats[chunk % 2], rows[chunk % 2], accs)
        if chunk + 2 < _NCHUNK:
            cps[chunk + 2] = fetch(chunk + 2)

    total = accs[0]
    for c in range(1, _NACC):
        total = total + accs[c]
    part_v[...] = total
    pltpu.sync_copy(part_v, out_hbm.at[wid])


@jax.jit
def kernel(features, labels, centers):
    lab32 = labels.astype(jnp.int32)
    partials = pl.kernel(
        _body,
        out_type=jax.ShapeDtypeStruct((_NW, _LANES), jnp.float32),
        mesh=plsc.VectorSubcoreMesh(core_axis_name="c", subcore_axis_name="s"),
        scratch_types=[
            pltpu.VMEM((_BPW,), jnp.int32),
            pltpu.VMEM((_CHUNK, _DIM), jnp.float32),
            pltpu.VMEM((_CHUNK, _DIM), jnp.float32),
            pltpu.VMEM((_CHUNK, _DIM), jnp.float32),
            pltpu.VMEM((_CHUNK, _DIM), jnp.float32),
            pltpu.VMEM((_LANES,), jnp.float32),
            pltpu.SemaphoreType.DMA,
            pltpu.SemaphoreType.DMA,
        ],
    )(features, lab32, centers)
    return jnp.sum(partials) / (_BATCH * _DIM)


# speedup vs baseline: 1.1696x; 1.0150x over previous
import functools

import jax
import jax.numpy as jnp
from jax import lax
from jax.experimental import pallas as pl
from jax.experimental.pallas import tpu as pltpu
from jax.experimental.pallas import tpu_sc as plsc

_BATCH = 16384
_DIM = 128
_LANES = 16
_NC = 2
_NS = 16
_NW = _NC * _NS
_BPW = _BATCH // _NW
_CHUNK = 128
_NCHUNK = _BPW // _CHUNK
_UNROLL = 4
_NACC = 8


def _reduce_chunk(feat_v, rows_v, accs):
    @plsc.parallel_loop(0, _CHUNK, step=_UNROLL, unroll=2, carry=tuple(accs))
    def loop(r, accs):
        accs = list(accs)
        for u in range(_UNROLL):
            for c in range(_DIM // _LANES):
                f = feat_v[r + u, pl.ds(c * _LANES, _LANES)]
                cv = rows_v[r + u, pl.ds(c * _LANES, _LANES)]
                d = f - cv
                a = (u * (_DIM // _LANES) + c) % _NACC
                accs[a] = accs[a] + d * d
        return tuple(accs)

    return loop


def _body(feat_hbm, lab_hbm, cent_hbm, out_hbm,
          idx_v, feat0, feat1, rows0, rows1, part_v, sem0, sem1):
    wid = lax.axis_index("s") * _NC + lax.axis_index("c")
    base = wid * _BPW
    pltpu.sync_copy(lab_hbm.at[pl.ds(base, _BPW)], idx_v)

    feats = (feat0, feat1)
    rows = (rows0, rows1)
    sems = (sem0, sem1)

    def fetch(chunk):
        s = chunk % 2
        cp_c = pltpu.async_copy(
            cent_hbm.at[idx_v.at[pl.ds(chunk * _CHUNK, _CHUNK)]],
            rows[s], sems[s])
        cp_f = pltpu.async_copy(
            feat_hbm.at[pl.ds(base + chunk * _CHUNK, _CHUNK), :],
            feats[s], sems[s])
        return (cp_c, cp_f)

    cps = [None] * _NCHUNK
    cps[0] = fetch(0)
    cps[1] = fetch(1)

    accs = (jnp.zeros((_LANES,), jnp.float32),) * _NACC
    for chunk in range(_NCHUNK):
        cps[chunk][0].wait()
        cps[chunk][1].wait()
        accs = _reduce_chunk(feats[chunk % 2], rows[chunk % 2], accs)
        if chunk + 2 < _NCHUNK:
            cps[chunk + 2] = fetch(chunk + 2)

    total = accs[0]
    for c in range(1, _NACC):
        total = total + accs[c]
    part_v[...] = total
    pltpu.sync_copy(part_v, out_hbm.at[wid])


@jax.jit
def kernel(features, labels, centers):
    lab32 = labels.astype(jnp.int32)
    partials = pl.kernel(
        _body,
        out_type=jax.ShapeDtypeStruct((_NW, _LANES), jnp.float32),
        mesh=plsc.VectorSubcoreMesh(core_axis_name="c", subcore_axis_name="s"),
        scratch_types=[
            pltpu.VMEM((_BPW,), jnp.int32),
            pltpu.VMEM((_CHUNK, _DIM), jnp.float32),
            pltpu.VMEM((_CHUNK, _DIM), jnp.float32),
            pltpu.VMEM((_CHUNK, _DIM), jnp.float32),
            pltpu.VMEM((_CHUNK, _DIM), jnp.float32),
            pltpu.VMEM((_LANES,), jnp.float32),
            pltpu.SemaphoreType.DMA,
            pltpu.SemaphoreType.DMA,
        ],
    )(features, lab32, centers)
    return jnp.sum(partials) / (_BATCH * _DIM)


# speedup vs baseline: 1.2412x; 1.0612x over previous
import functools

import jax
import jax.numpy as jnp
from jax import lax
from jax.experimental import pallas as pl
from jax.experimental.pallas import tpu as pltpu
from jax.experimental.pallas import tpu_sc as plsc

_BATCH = 16384
_DIM = 128
_LANES = 16
_NC = 2
_NS = 16
_NW = _NC * _NS
_BPW = _BATCH // _NW
_CHUNK = 128
_NCHUNK = _BPW // _CHUNK
_UNROLL = 4
_NACC = 8


def _body(feat_hbm, lab_hbm, cent_hbm, out_hbm,
          idx_v, feat_v, rows_v, part_v, semg, semf):
    wid = lax.axis_index("s") * _NC + lax.axis_index("c")
    base = wid * _BPW
    pltpu.sync_copy(lab_hbm.at[pl.ds(base, _BPW)], idx_v)

    def gather_desc(chunk, slot):
        return pltpu.make_async_copy(
            cent_hbm.at[idx_v.at[pl.ds(chunk * _CHUNK, _CHUNK)]],
            rows_v.at[slot], semg)

    def feat_desc(chunk):
        return pltpu.make_async_copy(
            feat_hbm.at[pl.ds(base + chunk * _CHUNK, _CHUNK), :],
            feat_v.at[pl.ds(chunk * _CHUNK, _CHUNK), :], semf)

    for j in range(_NCHUNK):
        feat_desc(j).start()
    gather_desc(0, 0).start()
    gather_desc(1, 1).start()

    def chunk_body(k, accs):
        for j in range(_NCHUNK):
            @pl.when(k == j)
            def _():
                feat_desc(j).wait()
                gather_desc(j, j % 2).wait()

        s = lax.rem(k, 2)
        fbase = k * _CHUNK

        @plsc.parallel_loop(0, _CHUNK, step=_UNROLL, unroll=2,
                            carry=tuple(accs))
        def loop(r, accs):
            accs = list(accs)
            for u in range(_UNROLL):
                for c in range(_DIM // _LANES):
                    f = feat_v[fbase + r + u, pl.ds(c * _LANES, _LANES)]
                    cv = rows_v[s, r + u, pl.ds(c * _LANES, _LANES)]
                    d = f - cv
                    a = (u * (_DIM // _LANES) + c) % _NACC
                    accs[a] = accs[a] + d * d
            return tuple(accs)

        for j in range(_NCHUNK - 2):
            @pl.when(k == j)
            def _():
                gather_desc(j + 2, j % 2).start()

        return loop

    accs = (jnp.zeros((_LANES,), jnp.float32),) * _NACC
    accs = lax.fori_loop(0, _NCHUNK, chunk_body, accs)

    total = accs[0]
    for c in range(1, _NACC):
        total = total + accs[c]
    part_v[...] = total
    pltpu.sync_copy(part_v, out_hbm.at[wid])


@jax.jit
def kernel(features, labels, centers):
    lab32 = labels.astype(jnp.int32)
    partials = pl.kernel(
        _body,
        out_type=jax.ShapeDtypeStruct((_NW, _LANES), jnp.float32),
        mesh=plsc.VectorSubcoreMesh(core_axis_name="c", subcore_axis_name="s"),
        scratch_types=[
            pltpu.VMEM((_BPW,), jnp.int32),
            pltpu.VMEM((_BPW, _DIM), jnp.float32),
            pltpu.VMEM((2, _CHUNK, _DIM), jnp.float32),
            pltpu.VMEM((_LANES,), jnp.float32),
            pltpu.SemaphoreType.DMA,
            pltpu.SemaphoreType.DMA,
        ],
    )(features, lab32, centers)
    return jnp.sum(partials) / (_BATCH * _DIM)


# speedup vs baseline: 1.2497x; 1.0068x over previous
import functools

import jax
import jax.numpy as jnp
from jax import lax
from jax.experimental import pallas as pl
from jax.experimental.pallas import tpu as pltpu
from jax.experimental.pallas import tpu_sc as plsc

_BATCH = 16384
_DIM = 128
_LANES = 16
_NC = 2
_NS = 16
_NW = _NC * _NS
_BPW = _BATCH // _NW
_CHUNK = 128
_NCHUNK = _BPW // _CHUNK
_UNROLL = 4
_NACC = 8


def _body(feat_hbm, lab_hbm, cent_hbm, out_hbm,
          idx_v, feat_v, rows_v, part_v, semg, semf):
    wid = lax.axis_index("s") * _NC + lax.axis_index("c")
    base = wid * _BPW

    def gather_desc(chunk, slot):
        return pltpu.make_async_copy(
            cent_hbm.at[idx_v.at[pl.ds(chunk * _CHUNK, _CHUNK)]],
            rows_v.at[slot], semg)

    def feat_desc(chunk):
        return pltpu.make_async_copy(
            feat_hbm.at[pl.ds(base + chunk * _CHUNK, _CHUNK), :],
            feat_v.at[pl.ds(chunk * _CHUNK, _CHUNK), :], semf)

    for j in range(_NCHUNK):
        feat_desc(j).start()
    pltpu.sync_copy(lab_hbm.at[pl.ds(base, _BPW)], idx_v)
    gather_desc(0, 0).start()
    gather_desc(1, 1).start()

    def chunk_body(k, accs):
        for j in range(_NCHUNK):
            @pl.when(k == j)
            def _():
                feat_desc(j).wait()
                gather_desc(j, j % 2).wait()

        s = lax.rem(k, 2)
        fbase = k * _CHUNK

        @plsc.parallel_loop(0, _CHUNK, step=_UNROLL, unroll=2,
                            carry=tuple(accs))
        def loop(r, accs):
            accs = list(accs)
            for u in range(_UNROLL):
                for c in range(_DIM // _LANES):
                    f = feat_v[fbase + r + u, pl.ds(c * _LANES, _LANES)]
                    cv = rows_v[s, r + u, pl.ds(c * _LANES, _LANES)]
                    d = f - cv
                    a = (u * (_DIM // _LANES) + c) % _NACC
                    accs[a] = accs[a] + d * d
            return tuple(accs)

        for j in range(_NCHUNK - 2):
            @pl.when(k == j)
            def _():
                gather_desc(j + 2, j % 2).start()

        return loop

    accs = (jnp.zeros((_LANES,), jnp.float32),) * _NACC
    accs = lax.fori_loop(0, _NCHUNK, chunk_body, accs)

    total = accs[0]
    for c in range(1, _NACC):
        total = total + accs[c]
    part_v[...] = total
    pltpu.sync_copy(part_v, out_hbm.at[wid])


@jax.jit
def kernel(features, labels, centers):
    lab32 = labels.astype(jnp.int32)
    partials = pl.kernel(
        _body,
        out_type=jax.ShapeDtypeStruct((_NW, _LANES), jnp.float32),
        mesh=plsc.VectorSubcoreMesh(core_axis_name="c", subcore_axis_name="s"),
        scratch_types=[
            pltpu.VMEM((_BPW,), jnp.int32),
            pltpu.VMEM((_BPW, _DIM), jnp.float32),
            pltpu.VMEM((2, _CHUNK, _DIM), jnp.float32),
            pltpu.VMEM((_LANES,), jnp.float32),
            pltpu.SemaphoreType.DMA,
            pltpu.SemaphoreType.DMA,
        ],
    )(features, lab32, centers)
    return jnp.sum(partials) / (_BATCH * _DIM)
